# Initial kernel scaffold; baseline (speedup 1.0000x reference)
import jax
import jax.numpy as jnp
from jax.experimental import pallas as pl


def kernel(vocab, type, vocab_table, type_table):
    raise NotImplementedError("write your pallas kernel here")


# speedup vs baseline: 4.3433x; 4.3433x over previous
import functools

import jax
import jax.numpy as jnp
from jax import lax
from jax.experimental import pallas as pl
from jax.experimental.pallas import tpu as pltpu
from jax.experimental.pallas import tpu_sc as plsc

_HIDDEN = 128
_GROUP = 128


def _emb_kernel(n_tokens, n_workers, groups_per_worker):
    mesh = plsc.VectorSubcoreMesh(core_axis_name="c", subcore_axis_name="s")

    @functools.partial(
        pl.kernel,
        mesh=mesh,
        out_type=jax.ShapeDtypeStruct((n_tokens, _HIDDEN), jnp.float32),
        scratch_types=[
            pltpu.VMEM((_GROUP,), jnp.int32),
            pltpu.VMEM((_GROUP,), jnp.int32),
            pltpu.VMEM((_GROUP, _HIDDEN), jnp.float32),
            pltpu.VMEM((2, _HIDDEN), jnp.float32),
            pltpu.SemaphoreType.DMA,
        ],
    )
    def body(vidx_hbm, tidx_hbm, vtab_hbm, ttab_hbm, out_hbm,
             vidx_v, tidx_v, rows_v, ttab_v, sem):
        wid = lax.axis_index("s") * 2 + lax.axis_index("c")
        base = wid * (groups_per_worker * _GROUP)
        pltpu.sync_copy(ttab_hbm, ttab_v)
        t0 = [ttab_v[0, pl.ds(16 * j, 16)] for j in range(_HIDDEN // 16)]
        dt = [ttab_v[1, pl.ds(16 * j, 16)] - t0[j]
              for j in range(_HIDDEN // 16)]

        def group(g, carry):
            off = base + g * _GROUP
            pltpu.sync_copy(vidx_hbm.at[pl.ds(off, _GROUP)], vidx_v)
            pltpu.sync_copy(tidx_hbm.at[pl.ds(off, _GROUP)], tidx_v)
            pltpu.async_copy(vtab_hbm.at[vidx_v], rows_v, sem).wait()

            def block(blk, c2):
                tv = tidx_v[pl.ds(16 * blk, 16)].astype(jnp.float32)
                for k in range(16):
                    r = 16 * blk + k
                    ts = jnp.broadcast_to(tv[k], (16,))
                    for j in range(_HIDDEN // 16):
                        sl = pl.ds(16 * j, 16)
                        rows_v[r, sl] = (
                            rows_v[r, sl] + (t0[j] + ts * dt[j]))
                return c2

            lax.fori_loop(0, _GROUP // 16, block, 0)
            pltpu.sync_copy(rows_v, out_hbm.at[pl.ds(off, _GROUP)])
            return carry

        lax.fori_loop(0, groups_per_worker, group, 0)

    return body


def kernel(vocab, type, vocab_table, type_table):
    b, l = vocab.shape
    n_tokens = b * l
    info = plsc.get_sparse_core_info()
    n_workers = info.num_cores * info.num_subcores
    groups_per_worker = n_tokens // (n_workers * _GROUP)
    vidx = vocab.reshape(n_tokens)
    tidx = type.reshape(n_tokens)
    out = _emb_kernel(n_tokens, n_workers, groups_per_worker)(
        vidx, tidx, vocab_table, type_table)
    return out.reshape(b, l, _HIDDEN)
